# Initial kernel scaffold; baseline (speedup 1.0000x reference)
import functools

import jax
import jax.numpy as jnp
from jax import lax
from jax.experimental import pallas as pl
from jax.experimental.pallas import tpu as pltpu
from jax.experimental.pallas import tpu_sc as plsc

VOCAB = 671
D = 768
SEQ = 7
BATCH = 16384
ROWS = BATCH * SEQ
NW = 32
R_PER_TILE = ROWS // NW
CHUNK = 64
NCH = R_PER_TILE // CHUNK


def _prep_body(tok_ref, pos_ref, gamma_ref, beta_ref, out_ref):
    emb = tok_ref[...] + pos_ref[...]
    mean = jnp.mean(emb, axis=-1, keepdims=True)
    cen = emb - mean
    var = jnp.mean(cen * cen, axis=-1, keepdims=True)
    normed = cen * lax.rsqrt(var + 1e-5)
    out_ref[...] = (normed * gamma_ref[...] + beta_ref[...])[None]


_prep = pl.pallas_call(
    _prep_body,
    grid=(SEQ,),
    in_specs=[
        pl.BlockSpec((VOCAB, D), lambda s: (0, 0)),
        pl.BlockSpec((1, D), lambda s: (s, 0)),
        pl.BlockSpec((1, D), lambda s: (0, 0)),
        pl.BlockSpec((1, D), lambda s: (0, 0)),
    ],
    out_specs=pl.BlockSpec((1, VOCAB, D), lambda s: (s, 0, 0)),
    out_shape=jax.ShapeDtypeStruct((SEQ, VOCAB, D), jnp.float32),
)


def _make_sc_gather():
    mesh = plsc.VectorSubcoreMesh(core_axis_name="c", subcore_axis_name="s")

    @functools.partial(
        pl.kernel,
        mesh=mesh,
        out_type=jax.ShapeDtypeStruct((ROWS, D), jnp.float32),
        scratch_types=[
            pltpu.VMEM((NCH, CHUNK), jnp.int32),
            pltpu.VMEM((CHUNK, D), jnp.float32),
            pltpu.SemaphoreType.DMA,
            pltpu.SemaphoreType.DMA,
        ],
    )
    def k(table_hbm, idx_hbm, out_hbm, idx_v, buf, gsem, ssem):
        wid = lax.axis_index("s") * 2 + lax.axis_index("c")
        base = wid * R_PER_TILE
        pltpu.sync_copy(idx_hbm.at[wid], idx_v)

        def body(j, carry):
            pltpu.async_copy(table_hbm.at[idx_v.at[j]], buf, gsem).wait()
            pltpu.async_copy(
                buf, out_hbm.at[pl.ds(base + j * CHUNK, CHUNK)], ssem
            ).wait()
            return carry

        lax.fori_loop(0, NCH, body, 0)

    return k


_sc_gather = _make_sc_gather()


def kernel(x, tok_table, pos_table, gamma, beta):
    combined = _prep(
        tok_table, pos_table, gamma.reshape(1, D), beta.reshape(1, D)
    )
    flat_table = combined.reshape(SEQ * VOCAB, D)
    idx = (
        x.astype(jnp.int32) + jnp.arange(SEQ, dtype=jnp.int32)[None, :] * VOCAB
    ).reshape(NW, NCH, CHUNK)
    out = _sc_gather(flat_table, idx)
    return out.reshape(BATCH, SEQ, D)


# speedup vs baseline: 2.0037x; 2.0037x over previous
import functools

import jax
import jax.numpy as jnp
from jax import lax
from jax.experimental import pallas as pl
from jax.experimental.pallas import tpu as pltpu
from jax.experimental.pallas import tpu_sc as plsc

VOCAB = 671
D = 768
SEQ = 7
BATCH = 16384
ROWS = BATCH * SEQ
NW = 32
R_PER_TILE = ROWS // NW
CHUNK = 64
NCH = R_PER_TILE // CHUNK


def _prep_body(tok_ref, pos_ref, gamma_ref, beta_ref, out_ref):
    emb = tok_ref[...] + pos_ref[0]
    mean = jnp.mean(emb, axis=-1, keepdims=True)
    cen = emb - mean
    var = jnp.mean(cen * cen, axis=-1, keepdims=True)
    normed = cen * lax.rsqrt(var + 1e-5)
    out_ref[...] = (normed * gamma_ref[...] + beta_ref[...])[None]


_prep = pl.pallas_call(
    _prep_body,
    grid=(SEQ,),
    in_specs=[
        pl.BlockSpec((VOCAB, D), lambda s: (0, 0)),
        pl.BlockSpec((1, 1, D), lambda s: (s, 0, 0)),
        pl.BlockSpec((1, D), lambda s: (0, 0)),
        pl.BlockSpec((1, D), lambda s: (0, 0)),
    ],
    out_specs=pl.BlockSpec((1, VOCAB, D), lambda s: (s, 0, 0)),
    out_shape=jax.ShapeDtypeStruct((SEQ, VOCAB, D), jnp.float32),
)


def _make_sc_gather():
    mesh = plsc.VectorSubcoreMesh(core_axis_name="c", subcore_axis_name="s")

    @functools.partial(
        pl.kernel,
        mesh=mesh,
        out_type=jax.ShapeDtypeStruct((ROWS, D), jnp.float32),
        scratch_types=[
            pltpu.VMEM((NCH, CHUNK), jnp.int32),
            pltpu.VMEM((CHUNK, D), jnp.float32),
            pltpu.SemaphoreType.DMA,
            pltpu.SemaphoreType.DMA,
        ],
    )
    def k(table_hbm, idx_hbm, out_hbm, idx_v, buf, gsem, ssem):
        wid = lax.axis_index("s") * 2 + lax.axis_index("c")
        base = wid * R_PER_TILE
        pltpu.sync_copy(idx_hbm.at[wid], idx_v)

        def body(j, carry):
            pltpu.async_copy(table_hbm.at[idx_v.at[j]], buf, gsem).wait()
            pltpu.async_copy(
                buf, out_hbm.at[pl.ds(base + j * CHUNK, CHUNK)], ssem
            ).wait()
            return carry

        lax.fori_loop(0, NCH, body, 0)

    return k


_sc_gather = _make_sc_gather()


def kernel(x, tok_table, pos_table, gamma, beta):
    combined = _prep(
        tok_table,
        pos_table.reshape(SEQ, 1, D),
        gamma.reshape(1, D),
        beta.reshape(1, D),
    )
    flat_table = combined.reshape(SEQ * VOCAB, D)
    idx = (
        x.astype(jnp.int32) + jnp.arange(SEQ, dtype=jnp.int32)[None, :] * VOCAB
    ).reshape(NW, NCH, CHUNK)
    out = _sc_gather(flat_table, idx)
    return out.reshape(BATCH, SEQ, D)


# speedup vs baseline: 2.0773x; 1.0367x over previous
import functools

import jax
import jax.numpy as jnp
from jax import lax
from jax.experimental import pallas as pl
from jax.experimental.pallas import tpu as pltpu
from jax.experimental.pallas import tpu_sc as plsc

VOCAB = 671
D = 768
SEQ = 7
BATCH = 16384
ROWS = BATCH * SEQ
NW = 32
R_PER_TILE = ROWS // NW
CHUNK = 64
NCH = R_PER_TILE // CHUNK


def _prep_body(tok_ref, pos_ref, gamma_ref, beta_ref, out_ref):
    emb = tok_ref[...] + pos_ref[0]
    mean = jnp.mean(emb, axis=-1, keepdims=True)
    cen = emb - mean
    var = jnp.mean(cen * cen, axis=-1, keepdims=True)
    normed = cen * lax.rsqrt(var + 1e-5)
    out_ref[...] = (normed * gamma_ref[...] + beta_ref[...])[None]


_prep = pl.pallas_call(
    _prep_body,
    grid=(SEQ,),
    in_specs=[
        pl.BlockSpec((VOCAB, D), lambda s: (0, 0)),
        pl.BlockSpec((1, 1, D), lambda s: (s, 0, 0)),
        pl.BlockSpec((1, D), lambda s: (0, 0)),
        pl.BlockSpec((1, D), lambda s: (0, 0)),
    ],
    out_specs=pl.BlockSpec((1, VOCAB, D), lambda s: (s, 0, 0)),
    out_shape=jax.ShapeDtypeStruct((SEQ, VOCAB, D), jnp.float32),
)


def _make_sc_gather():
    mesh = plsc.VectorSubcoreMesh(core_axis_name="c", subcore_axis_name="s")

    @functools.partial(
        pl.kernel,
        mesh=mesh,
        out_type=jax.ShapeDtypeStruct((ROWS, D), jnp.float32),
        scratch_types=[
            pltpu.VMEM((NCH, CHUNK), jnp.int32),
            pltpu.VMEM((CHUNK, D), jnp.float32),
            pltpu.VMEM((CHUNK, D), jnp.float32),
            pltpu.SemaphoreType.DMA,
            pltpu.SemaphoreType.DMA,
            pltpu.SemaphoreType.DMA,
            pltpu.SemaphoreType.DMA,
        ],
    )
    def k(table_hbm, idx_hbm, out_hbm, idx_v, buf0, buf1, g0, g1, s0, s1):
        wid = lax.axis_index("s") * 2 + lax.axis_index("c")
        base = wid * R_PER_TILE
        pltpu.sync_copy(idx_hbm.at[wid], idx_v)

        bufs = (buf0, buf1)
        gsems = (g0, g1)
        ssems = (s0, s1)

        def start_g(b, j):
            pltpu.async_copy(table_hbm.at[idx_v.at[j]], bufs[b], gsems[b])

        def wait_g(b):
            pltpu.make_async_copy(
                table_hbm.at[idx_v.at[0]], bufs[b], gsems[b]
            ).wait()

        def start_s(b, j):
            pltpu.async_copy(
                bufs[b], out_hbm.at[pl.ds(base + j * CHUNK, CHUNK)], ssems[b]
            )

        def wait_s(b):
            pltpu.make_async_copy(
                bufs[b], out_hbm.at[pl.ds(base, CHUNK)], ssems[b]
            ).wait()

        start_g(0, 0)
        wait_g(0)
        start_g(1, 1)
        start_s(0, 0)

        def group(g, carry):
            j1 = 2 * g + 1
            wait_g(1)
            wait_s(0)
            start_g(0, j1 + 1)
            start_s(1, j1)
            wait_g(0)
            wait_s(1)
            start_g(1, j1 + 2)
            start_s(0, j1 + 1)
            return carry

        lax.fori_loop(0, (NCH - 2) // 2, group, 0)

        wait_g(1)
        wait_s(0)
        start_s(1, NCH - 1)
        wait_s(1)

    return k


_sc_gather = _make_sc_gather()


def kernel(x, tok_table, pos_table, gamma, beta):
    combined = _prep(
        tok_table,
        pos_table.reshape(SEQ, 1, D),
        gamma.reshape(1, D),
        beta.reshape(1, D),
    )
    flat_table = combined.reshape(SEQ * VOCAB, D)
    idx = (
        x.astype(jnp.int32) + jnp.arange(SEQ, dtype=jnp.int32)[None, :] * VOCAB
    ).reshape(NW, NCH, CHUNK)
    out = _sc_gather(flat_table, idx)
    return out.reshape(BATCH, SEQ, D)


# speedup vs baseline: 6.4635x; 3.1115x over previous
import functools

import jax
import jax.numpy as jnp
from jax import lax
from jax.experimental import pallas as pl
from jax.experimental.pallas import tpu as pltpu
from jax.experimental.pallas import tpu_sc as plsc

VOCAB = 671
D = 768
SEQ = 7
BATCH = 16384
ROWS = BATCH * SEQ
NW = 32
R_PER_TILE = ROWS // NW
CHUNK = 64
NCH = R_PER_TILE // CHUNK


def _prep_body(tok_ref, pos_ref, gamma_ref, beta_ref, out_ref):
    emb = tok_ref[...] + pos_ref[0]
    mean = jnp.mean(emb, axis=-1, keepdims=True)
    cen = emb - mean
    var = jnp.mean(cen * cen, axis=-1, keepdims=True)
    normed = cen * lax.rsqrt(var + 1e-5)
    out_ref[...] = (normed * gamma_ref[...] + beta_ref[...])[None]


_prep = pl.pallas_call(
    _prep_body,
    grid=(SEQ,),
    in_specs=[
        pl.BlockSpec((VOCAB, D), lambda s: (0, 0)),
        pl.BlockSpec((1, 1, D), lambda s: (s, 0, 0)),
        pl.BlockSpec((1, D), lambda s: (0, 0)),
        pl.BlockSpec((1, D), lambda s: (0, 0)),
    ],
    out_specs=pl.BlockSpec((1, VOCAB, D), lambda s: (s, 0, 0)),
    out_shape=jax.ShapeDtypeStruct((SEQ, VOCAB, D), jnp.float32),
)


def _make_sc_gather():
    mesh = plsc.VectorSubcoreMesh(core_axis_name="c", subcore_axis_name="s")

    @functools.partial(
        pl.kernel,
        mesh=mesh,
        out_type=jax.ShapeDtypeStruct((ROWS, D), jnp.float32),
        scratch_types=[
            pltpu.VMEM((NCH, CHUNK), jnp.int32),
            pltpu.VMEM((CHUNK, D), jnp.float32),
            pltpu.VMEM((CHUNK, D), jnp.float32),
            pltpu.SemaphoreType.DMA,
            pltpu.SemaphoreType.DMA,
            pltpu.SemaphoreType.DMA,
            pltpu.SemaphoreType.DMA,
        ],
    )
    def k(table_hbm, idx_hbm, out_hbm, idx_v, buf0, buf1, g0, g1, s0, s1):
        wid = lax.axis_index("s") * 2 + lax.axis_index("c")
        base = wid * R_PER_TILE
        pltpu.sync_copy(idx_hbm.at[wid], idx_v)

        bufs = (buf0, buf1)
        gsems = (g0, g1)
        ssems = (s0, s1)

        def start_g(b, j):
            pltpu.async_copy(table_hbm.at[idx_v.at[j]], bufs[b], gsems[b])

        def wait_g(b):
            pltpu.make_async_copy(
                table_hbm.at[idx_v.at[0]], bufs[b], gsems[b]
            ).wait()

        def start_s(b, j):
            pltpu.async_copy(
                bufs[b], out_hbm.at[pl.ds(base + j * CHUNK, CHUNK)], ssems[b]
            )

        def wait_s(b):
            pltpu.make_async_copy(
                bufs[b], out_hbm.at[pl.ds(base, CHUNK)], ssems[b]
            ).wait()

        start_g(0, 0)
        wait_g(0)
        start_g(1, 1)
        start_s(0, 0)

        def group(g, carry):
            j1 = 2 * g + 1
            wait_g(1)
            wait_s(0)
            start_g(0, j1 + 1)
            start_s(1, j1)
            wait_g(0)
            wait_s(1)
            start_g(1, j1 + 2)
            start_s(0, j1 + 1)
            return carry

        lax.fori_loop(0, (NCH - 2) // 2, group, 0)

        wait_g(1)
        wait_s(0)
        start_s(1, NCH - 1)
        wait_s(1)

    return k


_sc_gather = _make_sc_gather()


def kernel(x, tok_table, pos_table, gamma, beta):
    combined = _prep(
        tok_table,
        pos_table.reshape(SEQ, 1, D),
        gamma.reshape(1, D),
        beta.reshape(1, D),
    )
    flat_table = combined.reshape(SEQ * VOCAB, D)
    idx = (
        x.astype(jnp.int32).T + jnp.arange(SEQ, dtype=jnp.int32)[:, None] * VOCAB
    ).reshape(NW, NCH, CHUNK)
    out = _sc_gather(flat_table, idx)
    return out.reshape(SEQ, BATCH, D).transpose(1, 0, 2)
